# Initial kernel scaffold; baseline (speedup 1.0000x reference)
import jax
import jax.numpy as jnp
from jax import lax
from jax.experimental import pallas as pl
from jax.experimental.pallas import tpu as pltpu
from jax.experimental.pallas import tpu_sc as plsc

B, S = 4096, 200
D = 32
TOT = B * S
NC, NS = 2, 16
NW = NC * NS
PER_W = TOT // NW
GROUP = 128
CHUNK_G = 8
CHUNK = GROUP * CHUNK_G
N_CHUNKS = PER_W // CHUNK
IDX_ROWS = PER_W // GROUP


def _emb_body(idx_hbm, wre_hbm, wim_hbm, ore_hbm, oim_hbm,
              idx_v, buf_re, buf_im, sem_re, sem_im):
    wid = lax.axis_index("s") * NC + lax.axis_index("c")
    row0 = wid * IDX_ROWS
    out0 = wid * PER_W

    pltpu.sync_copy(idx_hbm.at[pl.ds(row0, IDX_ROWS)], idx_v)

    def chunk(c, carry):
        ir0 = c * CHUNK_G
        waits = []
        for g in range(CHUNK_G):
            waits.append(pltpu.async_copy(
                wre_hbm.at[idx_v.at[ir0 + g]],
                buf_re.at[pl.ds(g * GROUP, GROUP)], sem_re))
            waits.append(pltpu.async_copy(
                wim_hbm.at[idx_v.at[ir0 + g]],
                buf_im.at[pl.ds(g * GROUP, GROUP)], sem_im))
        for w in waits:
            w.wait()
        obase = out0 + c * CHUNK
        pltpu.sync_copy(buf_re, ore_hbm.at[pl.ds(obase, CHUNK)])
        pltpu.sync_copy(buf_im, oim_hbm.at[pl.ds(obase, CHUNK)])
        return carry

    lax.fori_loop(0, N_CHUNKS, chunk, 0)


@jax.jit
def kernel(input_ids, W_re, W_im):
    idx2d = input_ids.reshape(TOT // GROUP, GROUP)
    mesh = plsc.VectorSubcoreMesh(core_axis_name="c", subcore_axis_name="s")
    out_re, out_im = pl.kernel(
        _emb_body,
        out_type=[
            jax.ShapeDtypeStruct((TOT, D), jnp.float32),
            jax.ShapeDtypeStruct((TOT, D), jnp.float32),
        ],
        mesh=mesh,
        scratch_types=[
            pltpu.VMEM((IDX_ROWS, GROUP), jnp.int32),
            pltpu.VMEM((CHUNK, D), jnp.float32),
            pltpu.VMEM((CHUNK, D), jnp.float32),
            pltpu.SemaphoreType.DMA,
            pltpu.SemaphoreType.DMA,
        ],
    )(idx2d, W_re, W_im)
    return (out_re.reshape(B, S, D), out_im.reshape(B, S, D))


# speedup vs baseline: 1.5785x; 1.5785x over previous
import jax
import jax.numpy as jnp
from jax import lax
from jax.experimental import pallas as pl
from jax.experimental.pallas import tpu as pltpu
from jax.experimental.pallas import tpu_sc as plsc

B, S = 4096, 200
D = 32
TOT = B * S
NC, NS = 2, 16
NW = NC * NS
PER_W = TOT // NW
GROUP = 128
CHUNK_G = 8
CHUNK = GROUP * CHUNK_G
N_CHUNKS = PER_W // CHUNK
IDX_ROWS = PER_W // GROUP


def _emb_body(idx_hbm, wre_hbm, wim_hbm, ore_hbm, oim_hbm,
              idx_v, buf_re, buf_im, sem_re, sem_im):
    wid = lax.axis_index("s") * NC + lax.axis_index("c")
    row0 = wid * IDX_ROWS
    out0 = wid * PER_W

    pltpu.sync_copy(idx_hbm.at[pl.ds(row0, IDX_ROWS)], idx_v)

    def chunk(c, carry):
        ir0 = c * CHUNK_G
        waits = []
        for g in range(CHUNK_G):
            waits.append(pltpu.async_copy(
                wre_hbm.at[idx_v.at[ir0 + g]],
                buf_re.at[pl.ds(g * GROUP, GROUP)], sem_re))
            waits.append(pltpu.async_copy(
                wim_hbm.at[idx_v.at[ir0 + g]],
                buf_im.at[pl.ds(g * GROUP, GROUP)], sem_im))
        for w in waits:
            w.wait()
        obase = out0 + c * CHUNK
        pltpu.sync_copy(buf_re, ore_hbm.at[pl.ds(obase, CHUNK)])
        pltpu.sync_copy(buf_im, oim_hbm.at[pl.ds(obase, CHUNK)])
        return carry

    lax.fori_loop(0, N_CHUNKS, chunk, 0)


@jax.jit
def kernel(input_ids, W_re, W_im):
    idx2d = input_ids.reshape(TOT // GROUP, GROUP)
    mesh = plsc.VectorSubcoreMesh(core_axis_name="c", subcore_axis_name="s")
    out_re, out_im = pl.kernel(
        _emb_body,
        out_type=[
            jax.ShapeDtypeStruct((TOT, D), jnp.float32),
            jax.ShapeDtypeStruct((TOT, D), jnp.float32),
        ],
        mesh=mesh,
        scratch_types=[
            pltpu.VMEM((IDX_ROWS, GROUP), jnp.int32),
            pltpu.VMEM((CHUNK, D), jnp.float32),
            pltpu.VMEM((CHUNK, D), jnp.float32),
            pltpu.SemaphoreType.DMA,
            pltpu.SemaphoreType.DMA,
        ],
        compiler_params=pltpu.CompilerParams(use_tc_tiling_on_sc=False),
    )(idx2d, W_re, W_im)
    return (out_re.reshape(B, S, D), out_im.reshape(B, S, D))


# speedup vs baseline: 1.5885x; 1.0063x over previous
import jax
import jax.numpy as jnp
from jax import lax
from jax.experimental import pallas as pl
from jax.experimental.pallas import tpu as pltpu
from jax.experimental.pallas import tpu_sc as plsc

B, S = 4096, 200
D = 32
TOT = B * S
NC, NS = 2, 16
NW = NC * NS
PER_W = TOT // NW
GROUP = 128
CHUNK_G = 2
CHUNK = GROUP * CHUNK_G
N_CHUNKS = PER_W // CHUNK
IDX_ROWS = PER_W // GROUP
NSLOT = 4


def _emb_body(idx_hbm, wre_hbm, wim_hbm, ore_hbm, oim_hbm,
              idx_v, bre, bim,
              gs0, gs1, gs2, gs3, ws0, ws1, ws2, ws3):
    gsems = (gs0, gs1, gs2, gs3)
    wsems = (ws0, ws1, ws2, ws3)
    wid = lax.axis_index("s") * NC + lax.axis_index("c")
    row0 = wid * IDX_ROWS
    out0 = wid * PER_W

    pltpu.sync_copy(idx_hbm.at[pl.ds(row0, IDX_ROWS)], idx_v)

    def g_copies(c, j, mk):
        ir0 = c * CHUNK_G
        out = []
        for g in range(CHUNK_G):
            for tbl, buf in ((wre_hbm, bre), (wim_hbm, bim)):
                out.append(mk(tbl.at[idx_v.at[ir0 + g]],
                              buf.at[j, pl.ds(g * GROUP, GROUP)], gsems[j]))
        return out

    def w_copies(c, j, mk):
        obase = out0 + c * CHUNK
        return [mk(bre.at[j], ore_hbm.at[pl.ds(obase, CHUNK)], wsems[j]),
                mk(bim.at[j], oim_hbm.at[pl.ds(obase, CHUNK)], wsems[j])]

    def fire_g(c, j):
        g_copies(c, j, pltpu.async_copy)

    def wait_g(c, j):
        for d in g_copies(c, j, pltpu.make_async_copy):
            d.wait()

    def fire_w(c, j):
        w_copies(c, j, pltpu.async_copy)

    def wait_w(c, j):
        for d in w_copies(c, j, pltpu.make_async_copy):
            d.wait()

    fire_g(0, 0)
    fire_g(1, 1)
    for c in (0, 1):
        wait_g(c, c)
        fire_w(c, c)
        fire_g(c + 2, c + 2)

    def steady(p, carry):
        base = NSLOT * p + 2
        for off in range(NSLOT):
            c = base + off
            j = (2 + off) % NSLOT
            wait_g(c, j)
            fire_w(c, j)
            jr = (2 + off + 2) % NSLOT
            wait_w(c - 2, jr)
            fire_g(c + 2, jr)
        return carry

    lax.fori_loop(0, (N_CHUNKS - 4) // NSLOT, steady, 0)

    for c in (N_CHUNKS - 2, N_CHUNKS - 1):
        j = c % NSLOT
        wait_g(c, j)
        fire_w(c, j)
    for c in (N_CHUNKS - 4, N_CHUNKS - 3, N_CHUNKS - 2, N_CHUNKS - 1):
        wait_w(c, c % NSLOT)


@jax.jit
def kernel(input_ids, W_re, W_im):
    idx2d = input_ids.reshape(TOT // GROUP, GROUP)
    mesh = plsc.VectorSubcoreMesh(core_axis_name="c", subcore_axis_name="s")
    out_re, out_im = pl.kernel(
        _emb_body,
        out_type=[
            jax.ShapeDtypeStruct((TOT, D), jnp.float32),
            jax.ShapeDtypeStruct((TOT, D), jnp.float32),
        ],
        mesh=mesh,
        scratch_types=[
            pltpu.VMEM((IDX_ROWS, GROUP), jnp.int32),
            pltpu.VMEM((NSLOT, CHUNK, D), jnp.float32),
            pltpu.VMEM((NSLOT, CHUNK, D), jnp.float32),
        ] + [pltpu.SemaphoreType.DMA] * 8,
        compiler_params=pltpu.CompilerParams(use_tc_tiling_on_sc=False),
    )(idx2d, W_re, W_im)
    return (out_re.reshape(B, S, D), out_im.reshape(B, S, D))
